# baseline (speedup 1.0000x reference)
import functools

import jax
import jax.numpy as jnp
from jax import lax
from jax.experimental import pallas as pl
from jax.experimental.pallas import tpu as pltpu
from jax.experimental.pallas import tpu_sc as plsc

B = 8
N = 512 * 512
NB = 16384
NBR = 128
NBP = NBR * 128
SHIFT = 14
NW = 32
PER_W = N * B // NW
CH = 8192
NCHUNK = PER_W // CH


def _prep_body(l_ref, t_ref, m_ref, p_ref, top_ref):
    for i in range(2):
        l = l_ref[i, 0]
        t = t_ref[i, 0]
        tpos = jnp.where(t > 0.5, 1.0, 0.0).astype(jnp.float32)
        e = 1.0 - l * (2.0 * tpos - 1.0)
        m = jnp.max(e)
        m_ref[i, 0, :] = jnp.broadcast_to(m, (128,))
        p_ref[i, 0, :] = jnp.broadcast_to(jnp.sum(tpos), (128,))
        top_ref[i, 0, :] = jnp.broadcast_to(
            lax.bitcast_convert_type(m, jnp.int32), (128,))


def _prepass(logits4, targets4):
    return pl.pallas_call(
        _prep_body,
        grid=(B // 2,),
        in_specs=[
            pl.BlockSpec((2, 1, 512, 512), lambda i: (i, 0, 0, 0)),
            pl.BlockSpec((2, 1, 512, 512), lambda i: (i, 0, 0, 0)),
        ],
        out_specs=[
            pl.BlockSpec((2, 1, 128), lambda i: (i, 0, 0)),
            pl.BlockSpec((2, 1, 128), lambda i: (i, 0, 0)),
            pl.BlockSpec((2, 1, 128), lambda i: (i, 0, 0)),
        ],
        out_shape=[
            jax.ShapeDtypeStruct((B, 1, 128), jnp.float32),
            jax.ShapeDtypeStruct((B, 1, 128), jnp.float32),
            jax.ShapeDtypeStruct((B, 1, 128), jnp.int32),
        ],
    )(logits4, targets4)


def _sc_hist_body(l_hbm, t_hbm, top_hbm, out_hbm,
                  lb0, lb1, tb0, tb1, topv, h0, h1, h2, sem0, sem1):
    cid = lax.axis_index("c")
    sid = lax.axis_index("s")
    img = cid * 4 + sid // 4
    part = sid % 4
    orow = img * 4 + part
    band0 = part * (NCHUNK * 16)

    zeros16 = jnp.zeros((16,), jnp.float32)

    @plsc.parallel_loop(0, NBP // 16, unroll=8)
    def _zero(j):
        h0[pl.ds(j * 16, 16)] = zeros16
        h1[pl.ds(j * 16, 16)] = zeros16
        h2[pl.ds(j * 16, 16)] = zeros16

    pltpu.sync_copy(top_hbm.at[img, 0, pl.ds(0, 16)], topv)
    top = topv[...]

    iota = lax.iota(jnp.int32, 16)
    nxt_idx = jnp.minimum(iota + 1, 15)
    not_last = iota < 15
    cnt_run = (iota + 1).astype(jnp.float32)
    neg_cnt_run = -cnt_run
    gdn = lax.GatherDimensionNumbers(
        offset_dims=(), collapsed_slice_dims=(0,), start_index_map=(0,))

    lbufs = (lb0, lb1)
    tbufs = (tb0, tb1)
    sems = (sem0, sem1)

    def _start(ci):
        rows = band0 + ci * 16
        lc = pltpu.async_copy(
            l_hbm.at[img, pl.ds(rows, 16), :], lbufs[ci % 2], sems[ci % 2])
        tc_ = pltpu.async_copy(
            t_hbm.at[img, pl.ds(rows, 16), :], tbufs[ci % 2], sems[ci % 2])
        return lc, tc_

    pend = _start(0)
    for ci in range(NCHUNK):
        lbuf = lbufs[ci % 2]
        tbuf = tbufs[ci % 2]
        nxt_pend = _start(ci + 1) if ci + 1 < NCHUNK else None
        pend[0].wait()
        pend[1].wait()
        pend = nxt_pend

        @plsc.parallel_loop(0, CH // 16, unroll=4)
        def _vec(v):
            r = v >> 5
            c = (v & 31) * 16
            l = lbuf[r, pl.ds(c, 16)]
            t = tbuf[r, pl.ds(c, 16)]
            lt = l * t
            e = (1.0 + l) - (lt + lt)
            es0 = jnp.maximum(e, 0.0)
            ebits = lax.bitcast_convert_type(es0, jnp.int32)
            d = lax.shift_right_arithmetic(top - ebits, SHIFT)
            bkt = jnp.minimum(jnp.maximum((NB - 1) - d, 0), NB - 1)
            bs, es = plsc.sort_key_val(bkt, es0)
            _, gs = plsc.sort_key_val(bkt, t)
            nxt = lax.gather(bs, nxt_idx[:, None], gdn, slice_sizes=(1,),
                             mode=lax.GatherScatterMode.PROMISE_IN_BOUNDS)
            brk = bs != nxt
            isend = brk | (~not_last)
            issub = brk & not_last
            ce = plsc.cumsum(es)
            cg = plsc.cumsum(gs)
            plsc.addupdate_scatter(h0, [bs], cnt_run, mask=isend)
            plsc.addupdate_scatter(h0, [nxt], neg_cnt_run, mask=issub)
            plsc.addupdate_scatter(h1, [bs], cg, mask=isend)
            plsc.addupdate_scatter(h1, [nxt], -cg, mask=issub)
            plsc.addupdate_scatter(h2, [bs], ce, mask=isend)
            plsc.addupdate_scatter(h2, [nxt], -ce, mask=issub)

    obase = orow * 3 * NBP
    pltpu.sync_copy(h0, out_hbm.at[pl.ds(obase, NBP)])
    pltpu.sync_copy(h1, out_hbm.at[pl.ds(obase + NBP, NBP)])
    pltpu.sync_copy(h2, out_hbm.at[pl.ds(obase + 2 * NBP, NBP)])


def _sc_hist(l3, t3, topflat):
    mesh = plsc.VectorSubcoreMesh(core_axis_name="c", subcore_axis_name="s")
    k = functools.partial(
        pl.kernel,
        mesh=mesh,
        compiler_params=pltpu.CompilerParams(needs_layout_passes=False),
        out_type=jax.ShapeDtypeStruct((NW * 3 * NBP,), jnp.float32),
        scratch_types=[
            pltpu.VMEM((16, 512), jnp.float32),
            pltpu.VMEM((16, 512), jnp.float32),
            pltpu.VMEM((16, 512), jnp.float32),
            pltpu.VMEM((16, 512), jnp.float32),
            pltpu.VMEM((16,), jnp.int32),
            pltpu.VMEM((NBP,), jnp.float32),
            pltpu.VMEM((NBP,), jnp.float32),
            pltpu.VMEM((NBP,), jnp.float32),
            pltpu.SemaphoreType.DMA,
            pltpu.SemaphoreType.DMA,
        ],
    )(_sc_hist_body)
    return k(l3, t3, topflat)


def _fin_body(h_ref, p_ref, m_ref, o_ref):
    f32 = jnp.float32
    img = pl.program_id(0)
    r = lax.broadcasted_iota(jnp.int32, (128, 128), 0)
    c = lax.broadcasted_iota(jnp.int32, (128, 128), 1)
    w_incl = (r >= c).astype(f32)
    w_strict = (r > c).astype(f32)

    cnt = (h_ref[0, 0] + h_ref[1, 0]) + (h_ref[2, 0] + h_ref[3, 0])
    cntp = (h_ref[0, 1] + h_ref[1, 1]) + (h_ref[2, 1] + h_ref[3, 1])
    sm = (h_ref[0, 2] + h_ref[1, 2]) + (h_ref[2, 2] + h_ref[3, 2])

    def suffix(x):
        s_in = jnp.dot(x, w_incl, preferred_element_type=f32)
        rowtot = s_in[:, :1]
        above = jnp.dot(w_strict.T, rowtot, preferred_element_type=f32)
        return s_in + above

    suf = suffix(cnt)
    sufp = suffix(cntp)
    a = suf - cnt
    ap = sufp - cntp
    p = p_ref[0, 0, 0]
    m = m_ref[0, 0, 0]
    tp = cntp
    tn = cnt - cntp
    sumpos = sm * cntp / jnp.maximum(cnt, 1.0)
    sumneg = sm - sumpos
    aa = p + a - ap
    contrib = (sumpos + sumneg * (p - ap - tp) / jnp.maximum(aa + tn, 1.0)
               ) / jnp.maximum(aa, 1.0)
    loss = jnp.sum(contrib)
    loss = jnp.where(p == 0.0, jnp.maximum(m, 0.0), loss) * (1.0 / B)

    @pl.when(img == 0)
    def _():
        o_ref[...] = jnp.zeros((1, 1), f32)

    o_ref[...] += jnp.broadcast_to(loss, (1, 1))


def _finalize(h4, pc, mc):
    return pl.pallas_call(
        _fin_body,
        grid=(B,),
        in_specs=[
            pl.BlockSpec((4, 3, NBR, 128), lambda i: (i, 0, 0, 0)),
            pl.BlockSpec((1, 1, 128), lambda i: (i, 0, 0)),
            pl.BlockSpec((1, 1, 128), lambda i: (i, 0, 0)),
        ],
        out_specs=pl.BlockSpec((1, 1), lambda i: (0, 0)),
        out_shape=jax.ShapeDtypeStruct((1, 1), jnp.float32),
    )(h4, pc, mc)


def kernel(logits, targets):
    l4 = logits.reshape(B, 1, 512, 512)
    t4 = targets.reshape(B, 1, 512, 512)
    mc, pc, top = _prepass(l4, t4)
    hists = _sc_hist(logits.reshape(B, 512, 512), targets.reshape(B, 512, 512),
                     top)
    h4 = hists.reshape(NW, 3, NBR, 128)
    out = _finalize(h4, pc, mc)
    return out.reshape(())


# speedup vs baseline: 1.0401x; 1.0401x over previous
import functools

import jax
import jax.numpy as jnp
from jax import lax
from jax.experimental import pallas as pl
from jax.experimental.pallas import tpu as pltpu
from jax.experimental.pallas import tpu_sc as plsc

B = 8
N = 512 * 512
NB = 16384
NBR = 128
NBP = NBR * 128
SHIFT = 14
NW = 32
PER_W = N * B // NW
CH = 8192
NCHUNK = PER_W // CH


def _prep_body(l_ref, t_ref, m_ref, p_ref, top_ref):
    for i in range(2):
        l = l_ref[i, 0]
        t = t_ref[i, 0]
        tpos = jnp.where(t > 0.5, 1.0, 0.0).astype(jnp.float32)
        e = 1.0 - l * (2.0 * tpos - 1.0)
        m = jnp.max(e)
        m_ref[i, 0, :] = jnp.broadcast_to(m, (128,))
        p_ref[i, 0, :] = jnp.broadcast_to(jnp.sum(tpos), (128,))
        top_ref[i, 0, :] = jnp.broadcast_to(
            lax.bitcast_convert_type(m, jnp.int32), (128,))


def _prepass(logits4, targets4):
    return pl.pallas_call(
        _prep_body,
        grid=(B // 2,),
        in_specs=[
            pl.BlockSpec((2, 1, 512, 512), lambda i: (i, 0, 0, 0)),
            pl.BlockSpec((2, 1, 512, 512), lambda i: (i, 0, 0, 0)),
        ],
        out_specs=[
            pl.BlockSpec((2, 1, 128), lambda i: (i, 0, 0)),
            pl.BlockSpec((2, 1, 128), lambda i: (i, 0, 0)),
            pl.BlockSpec((2, 1, 128), lambda i: (i, 0, 0)),
        ],
        out_shape=[
            jax.ShapeDtypeStruct((B, 1, 128), jnp.float32),
            jax.ShapeDtypeStruct((B, 1, 128), jnp.float32),
            jax.ShapeDtypeStruct((B, 1, 128), jnp.int32),
        ],
    )(logits4, targets4)


def _sc_hist_body(l_hbm, t_hbm, top_hbm, out_hbm,
                  lb0, lb1, tb0, tb1, topv, h0, h1, h2, sem0, sem1):
    cid = lax.axis_index("c")
    sid = lax.axis_index("s")
    img = cid * 4 + sid // 4
    part = sid % 4
    orow = img * 4 + part
    band0 = part * (NCHUNK * 16)

    zeros16 = jnp.zeros((16,), jnp.float32)

    @plsc.parallel_loop(0, NBP // 16, unroll=8)
    def _zero(j):
        h0[pl.ds(j * 16, 16)] = zeros16
        h1[pl.ds(j * 16, 16)] = zeros16
        h2[pl.ds(j * 16, 16)] = zeros16

    pltpu.sync_copy(top_hbm.at[img, 0, pl.ds(0, 16)], topv)
    top = topv[...]

    iota = lax.iota(jnp.int32, 16)
    nxt_idx = jnp.minimum(iota + 1, 15)
    not_last = iota < 15
    cnt_run = (iota + 1).astype(jnp.float32)
    neg_cnt_run = -cnt_run
    gdn = lax.GatherDimensionNumbers(
        offset_dims=(), collapsed_slice_dims=(0,), start_index_map=(0,))

    lbufs = (lb0, lb1)
    tbufs = (tb0, tb1)
    sems = (sem0, sem1)

    def _start(ci):
        rows = band0 + ci * 16
        lc = pltpu.async_copy(
            l_hbm.at[img, pl.ds(rows, 16), :], lbufs[ci % 2], sems[ci % 2])
        tc_ = pltpu.async_copy(
            t_hbm.at[img, pl.ds(rows, 16), :], tbufs[ci % 2], sems[ci % 2])
        return lc, tc_

    pend = _start(0)
    for ci in range(NCHUNK):
        lbuf = lbufs[ci % 2]
        tbuf = tbufs[ci % 2]
        nxt_pend = _start(ci + 1) if ci + 1 < NCHUNK else None
        pend[0].wait()
        pend[1].wait()
        pend = nxt_pend

        @plsc.parallel_loop(0, CH // 16, unroll=4)
        def _vec(v):
            r = v >> 5
            c = (v & 31) * 16
            l = lbuf[r, pl.ds(c, 16)]
            t = tbuf[r, pl.ds(c, 16)]
            lt = l * t
            e = (1.0 + l) - (lt + lt)
            es0 = jnp.maximum(e, 0.0)
            ebits = lax.bitcast_convert_type(es0, jnp.int32)
            d = lax.shift_right_arithmetic(top - ebits, SHIFT)
            bkt = jnp.minimum(jnp.maximum((NB - 1) - d, 0), NB - 1)
            bs, es = plsc.sort_key_val(bkt, es0)
            _, gs = plsc.sort_key_val(bkt, t)
            nxt = lax.gather(bs, nxt_idx[:, None], gdn, slice_sizes=(1,),
                             mode=lax.GatherScatterMode.PROMISE_IN_BOUNDS)
            brk = bs != nxt
            isend = brk | (~not_last)
            issub = brk & not_last
            ce = plsc.cumsum(es)
            cg = plsc.cumsum(gs)
            plsc.addupdate_scatter(h0, [bs], cnt_run, mask=isend)
            plsc.addupdate_scatter(h0, [nxt], neg_cnt_run, mask=issub)
            plsc.addupdate_scatter(h1, [bs], cg, mask=isend)
            plsc.addupdate_scatter(h1, [nxt], -cg, mask=issub)
            plsc.addupdate_scatter(h2, [bs], ce, mask=isend)
            plsc.addupdate_scatter(h2, [nxt], -ce, mask=issub)

    obase = orow * 3 * NBP
    pltpu.sync_copy(h0, out_hbm.at[pl.ds(obase, NBP)])
    pltpu.sync_copy(h1, out_hbm.at[pl.ds(obase + NBP, NBP)])
    pltpu.sync_copy(h2, out_hbm.at[pl.ds(obase + 2 * NBP, NBP)])


def _sc_hist(l3, t3, topflat):
    mesh = plsc.VectorSubcoreMesh(core_axis_name="c", subcore_axis_name="s")
    k = functools.partial(
        pl.kernel,
        mesh=mesh,
        compiler_params=pltpu.CompilerParams(needs_layout_passes=False),
        out_type=jax.ShapeDtypeStruct((NW * 3 * NBP,), jnp.float32),
        scratch_types=[
            pltpu.VMEM((16, 512), jnp.float32),
            pltpu.VMEM((16, 512), jnp.float32),
            pltpu.VMEM((16, 512), jnp.float32),
            pltpu.VMEM((16, 512), jnp.float32),
            pltpu.VMEM((16,), jnp.int32),
            pltpu.VMEM((NBP,), jnp.float32),
            pltpu.VMEM((NBP,), jnp.float32),
            pltpu.VMEM((NBP,), jnp.float32),
            pltpu.SemaphoreType.DMA,
            pltpu.SemaphoreType.DMA,
        ],
    )(_sc_hist_body)
    return k(l3, t3, topflat)


def _fin_body(h_ref, p_ref, m_ref, o_ref):
    f32 = jnp.float32
    r = lax.broadcasted_iota(jnp.int32, (128, 128), 0)
    c = lax.broadcasted_iota(jnp.int32, (128, 128), 1)
    w_incl = (r >= c).astype(f32)
    w_strict = (r > c).astype(f32)

    total = jnp.zeros((), f32)
    for img in range(B):
        r0 = img * 4
        cnt = (h_ref[r0, 0] + h_ref[r0 + 1, 0]) + (h_ref[r0 + 2, 0] + h_ref[r0 + 3, 0])
        cntp = (h_ref[r0, 1] + h_ref[r0 + 1, 1]) + (h_ref[r0 + 2, 1] + h_ref[r0 + 3, 1])
        sm = (h_ref[r0, 2] + h_ref[r0 + 1, 2]) + (h_ref[r0 + 2, 2] + h_ref[r0 + 3, 2])

        def suffix(x):
            s_in = jnp.dot(x, w_incl, preferred_element_type=f32)
            rowtot = s_in[:, :1]
            above = jnp.dot(w_strict.T, rowtot, preferred_element_type=f32)
            return s_in + above

        suf = suffix(cnt)
        sufp = suffix(cntp)
        a = suf - cnt
        ap = sufp - cntp
        p = p_ref[img, 0, 0]
        m = m_ref[img, 0, 0]
        tp = cntp
        tn = cnt - cntp
        sumpos = sm * cntp / jnp.maximum(cnt, 1.0)
        sumneg = sm - sumpos
        aa = p + a - ap
        contrib = (sumpos + sumneg * (p - ap - tp) / jnp.maximum(aa + tn, 1.0)
                   ) / jnp.maximum(aa, 1.0)
        loss = jnp.sum(contrib)
        loss = jnp.where(p == 0.0, jnp.maximum(m, 0.0), loss)
        total = total + loss
    o_ref[...] = jnp.broadcast_to(total * (1.0 / B), (1, 1))


def _finalize(h4, pc, mc):
    return pl.pallas_call(
        _fin_body,
        in_specs=[
            pl.BlockSpec((NW, 3, NBR, 128), lambda: (0, 0, 0, 0)),
            pl.BlockSpec((B, 1, 128), lambda: (0, 0, 0)),
            pl.BlockSpec((B, 1, 128), lambda: (0, 0, 0)),
        ],
        out_specs=pl.BlockSpec((1, 1), lambda: (0, 0)),
        out_shape=jax.ShapeDtypeStruct((1, 1), jnp.float32),
    )(h4, pc, mc)


def kernel(logits, targets):
    l4 = logits.reshape(B, 1, 512, 512)
    t4 = targets.reshape(B, 1, 512, 512)
    mc, pc, top = _prepass(l4, t4)
    hists = _sc_hist(logits.reshape(B, 512, 512), targets.reshape(B, 512, 512),
                     top)
    h4 = hists.reshape(NW, 3, NBR, 128)
    out = _finalize(h4, pc, mc)
    return out.reshape(())


# speedup vs baseline: 1.0414x; 1.0013x over previous
import functools

import jax
import jax.numpy as jnp
from jax import lax
from jax.experimental import pallas as pl
from jax.experimental.pallas import tpu as pltpu
from jax.experimental.pallas import tpu_sc as plsc

B = 8
N = 512 * 512
NB = 16384
NBR = 128
NBP = NBR * 128
SHIFT = 14
NW = 32
PER_W = N * B // NW
CH = 8192
NCHUNK = PER_W // CH


def _prep_body(l_ref, t_ref, m_ref, p_ref, top_ref):
    for i in range(2):
        l = l_ref[i, 0]
        t = t_ref[i, 0]
        tpos = jnp.where(t > 0.5, 1.0, 0.0).astype(jnp.float32)
        e = 1.0 - l * (2.0 * tpos - 1.0)
        m = jnp.max(e)
        m_ref[i, 0, :] = jnp.broadcast_to(m, (128,))
        p_ref[i, 0, :] = jnp.broadcast_to(jnp.sum(tpos), (128,))
        top_ref[i, 0, :] = jnp.broadcast_to(
            lax.bitcast_convert_type(m, jnp.int32), (128,))


def _prepass(logits4, targets4):
    return pl.pallas_call(
        _prep_body,
        grid=(B // 2,),
        in_specs=[
            pl.BlockSpec((2, 1, 512, 512), lambda i: (i, 0, 0, 0)),
            pl.BlockSpec((2, 1, 512, 512), lambda i: (i, 0, 0, 0)),
        ],
        out_specs=[
            pl.BlockSpec((2, 1, 128), lambda i: (i, 0, 0)),
            pl.BlockSpec((2, 1, 128), lambda i: (i, 0, 0)),
            pl.BlockSpec((2, 1, 128), lambda i: (i, 0, 0)),
        ],
        out_shape=[
            jax.ShapeDtypeStruct((B, 1, 128), jnp.float32),
            jax.ShapeDtypeStruct((B, 1, 128), jnp.float32),
            jax.ShapeDtypeStruct((B, 1, 128), jnp.int32),
        ],
    )(logits4, targets4)


def _sc_hist_body(l_hbm, t_hbm, top_hbm, out_hbm,
                  lb0, lb1, tb0, tb1, topv, h0, h1, h2, sem0, sem1):
    cid = lax.axis_index("c")
    sid = lax.axis_index("s")
    img = cid * 4 + sid // 4
    part = sid % 4
    orow = img * 4 + part
    band0 = part * (NCHUNK * 16)

    zeros16 = jnp.zeros((16,), jnp.float32)

    @plsc.parallel_loop(0, NBP // 16, unroll=8)
    def _zero(j):
        h0[pl.ds(j * 16, 16)] = zeros16
        h1[pl.ds(j * 16, 16)] = zeros16
        h2[pl.ds(j * 16, 16)] = zeros16

    pltpu.sync_copy(top_hbm.at[img, 0, pl.ds(0, 16)], topv)
    top = topv[...]

    iota = lax.iota(jnp.int32, 16)
    nxt_idx = jnp.minimum(iota + 1, 15)
    not_last = iota < 15
    cnt_run = (iota + 1).astype(jnp.float32)
    neg_cnt_run = -cnt_run
    gdn = lax.GatherDimensionNumbers(
        offset_dims=(), collapsed_slice_dims=(0,), start_index_map=(0,))

    lbufs = (lb0, lb1)
    tbufs = (tb0, tb1)
    sems = (sem0, sem1)

    def _start(ci):
        rows = band0 + ci * 16
        lc = pltpu.async_copy(
            l_hbm.at[img, pl.ds(rows, 16), :], lbufs[ci % 2], sems[ci % 2])
        tc_ = pltpu.async_copy(
            t_hbm.at[img, pl.ds(rows, 16), :], tbufs[ci % 2], sems[ci % 2])
        return lc, tc_

    pend = _start(0)
    for ci in range(NCHUNK):
        lbuf = lbufs[ci % 2]
        tbuf = tbufs[ci % 2]
        nxt_pend = _start(ci + 1) if ci + 1 < NCHUNK else None
        pend[0].wait()
        pend[1].wait()
        pend = nxt_pend

        @plsc.parallel_loop(0, CH // 16, unroll=4)
        def _vec(v):
            r = v >> 5
            c = (v & 31) * 16
            l = lbuf[r, pl.ds(c, 16)]
            t = tbuf[r, pl.ds(c, 16)]
            lt = l * t
            e = (1.0 + l) - (lt + lt)
            es0 = jnp.maximum(e, 0.0)
            ebits = lax.bitcast_convert_type(es0, jnp.int32)
            d = lax.shift_right_arithmetic(top - ebits, SHIFT)
            bkt = jnp.minimum(jnp.maximum((NB - 1) - d, 0), NB - 1)
            bs, es = plsc.sort_key_val(bkt, es0)
            _, gs = plsc.sort_key_val(bkt, t)
            nxt = lax.gather(bs, nxt_idx[:, None], gdn, slice_sizes=(1,),
                             mode=lax.GatherScatterMode.PROMISE_IN_BOUNDS)
            brk = bs != nxt
            isend = brk | (~not_last)
            issub = brk & not_last
            ce = plsc.cumsum(es)
            cg = plsc.cumsum(gs)
            plsc.addupdate_scatter(h0, [bs], cnt_run, mask=isend)
            plsc.addupdate_scatter(h0, [nxt], neg_cnt_run, mask=issub)
            plsc.addupdate_scatter(h1, [bs], cg, mask=isend)
            plsc.addupdate_scatter(h1, [nxt], -cg, mask=issub)
            plsc.addupdate_scatter(h2, [bs], ce, mask=isend)
            plsc.addupdate_scatter(h2, [nxt], -ce, mask=issub)

    obase = orow * 3 * NBP
    o0 = pltpu.async_copy(h0, out_hbm.at[pl.ds(obase, NBP)], sem0)
    o1 = pltpu.async_copy(h1, out_hbm.at[pl.ds(obase + NBP, NBP)], sem1)
    o2 = pltpu.async_copy(h2, out_hbm.at[pl.ds(obase + 2 * NBP, NBP)], sem0)
    o0.wait()
    o1.wait()
    o2.wait()


def _sc_hist(l3, t3, topflat):
    mesh = plsc.VectorSubcoreMesh(core_axis_name="c", subcore_axis_name="s")
    k = functools.partial(
        pl.kernel,
        mesh=mesh,
        compiler_params=pltpu.CompilerParams(needs_layout_passes=False),
        out_type=jax.ShapeDtypeStruct((NW * 3 * NBP,), jnp.float32),
        scratch_types=[
            pltpu.VMEM((16, 512), jnp.float32),
            pltpu.VMEM((16, 512), jnp.float32),
            pltpu.VMEM((16, 512), jnp.float32),
            pltpu.VMEM((16, 512), jnp.float32),
            pltpu.VMEM((16,), jnp.int32),
            pltpu.VMEM((NBP,), jnp.float32),
            pltpu.VMEM((NBP,), jnp.float32),
            pltpu.VMEM((NBP,), jnp.float32),
            pltpu.SemaphoreType.DMA,
            pltpu.SemaphoreType.DMA,
        ],
    )(_sc_hist_body)
    return k(l3, t3, topflat)


def _fin_body(h_ref, p_ref, m_ref, o_ref):
    f32 = jnp.float32
    r = lax.broadcasted_iota(jnp.int32, (128, 128), 0)
    c = lax.broadcasted_iota(jnp.int32, (128, 128), 1)
    w_incl = (r >= c).astype(f32)
    w_strict = (r > c).astype(f32)

    total = jnp.zeros((), f32)
    for img in range(B):
        r0 = img * 4
        cnt = (h_ref[r0, 0] + h_ref[r0 + 1, 0]) + (h_ref[r0 + 2, 0] + h_ref[r0 + 3, 0])
        cntp = (h_ref[r0, 1] + h_ref[r0 + 1, 1]) + (h_ref[r0 + 2, 1] + h_ref[r0 + 3, 1])
        sm = (h_ref[r0, 2] + h_ref[r0 + 1, 2]) + (h_ref[r0 + 2, 2] + h_ref[r0 + 3, 2])

        def suffix(x):
            s_in = jnp.dot(x, w_incl, preferred_element_type=f32)
            rowtot = s_in[:, :1]
            above = jnp.dot(w_strict.T, rowtot, preferred_element_type=f32)
            return s_in + above

        suf = suffix(cnt)
        sufp = suffix(cntp)
        a = suf - cnt
        ap = sufp - cntp
        p = p_ref[img, 0, 0]
        m = m_ref[img, 0, 0]
        tp = cntp
        tn = cnt - cntp
        sumpos = sm * cntp / jnp.maximum(cnt, 1.0)
        sumneg = sm - sumpos
        aa = p + a - ap
        contrib = (sumpos + sumneg * (p - ap - tp) / jnp.maximum(aa + tn, 1.0)
                   ) / jnp.maximum(aa, 1.0)
        loss = jnp.sum(contrib)
        loss = jnp.where(p == 0.0, jnp.maximum(m, 0.0), loss)
        total = total + loss
    o_ref[...] = jnp.broadcast_to(total * (1.0 / B), (1, 1))


def _finalize(h4, pc, mc):
    return pl.pallas_call(
        _fin_body,
        in_specs=[
            pl.BlockSpec((NW, 3, NBR, 128), lambda: (0, 0, 0, 0)),
            pl.BlockSpec((B, 1, 128), lambda: (0, 0, 0)),
            pl.BlockSpec((B, 1, 128), lambda: (0, 0, 0)),
        ],
        out_specs=pl.BlockSpec((1, 1), lambda: (0, 0)),
        out_shape=jax.ShapeDtypeStruct((1, 1), jnp.float32),
    )(h4, pc, mc)


def kernel(logits, targets):
    l4 = logits.reshape(B, 1, 512, 512)
    t4 = targets.reshape(B, 1, 512, 512)
    mc, pc, top = _prepass(l4, t4)
    hists = _sc_hist(logits.reshape(B, 512, 512), targets.reshape(B, 512, 512),
                     top)
    h4 = hists.reshape(NW, 3, NBR, 128)
    out = _finalize(h4, pc, mc)
    return out.reshape(())


# speedup vs baseline: 1.0693x; 1.0267x over previous
import functools

import jax
import jax.numpy as jnp
from jax import lax
from jax.experimental import pallas as pl
from jax.experimental.pallas import tpu as pltpu
from jax.experimental.pallas import tpu_sc as plsc

B = 8
N = 512 * 512
NB = 16384
NBR = 128
NBP = NBR * 128
SHIFT = 14
NW = 32
PER_W = N * B // NW
CH = 8192
NCHUNK = PER_W // CH


def _prep_body(l_ref, t_ref, m_ref, p_ref, top_ref):
    for i in range(2):
        l = l_ref[i, 0]
        t = t_ref[i, 0]
        tpos = jnp.where(t > 0.5, 1.0, 0.0).astype(jnp.float32)
        e = 1.0 - l * (2.0 * tpos - 1.0)
        m = jnp.max(e)
        m_ref[i, 0, :] = jnp.broadcast_to(m, (128,))
        p_ref[i, 0, :] = jnp.broadcast_to(jnp.sum(tpos), (128,))
        top_ref[i, 0, :] = jnp.broadcast_to(
            lax.bitcast_convert_type(m, jnp.int32), (128,))


def _prepass(logits4, targets4):
    return pl.pallas_call(
        _prep_body,
        grid=(B // 2,),
        in_specs=[
            pl.BlockSpec((2, 1, 512, 512), lambda i: (i, 0, 0, 0)),
            pl.BlockSpec((2, 1, 512, 512), lambda i: (i, 0, 0, 0)),
        ],
        out_specs=[
            pl.BlockSpec((2, 1, 128), lambda i: (i, 0, 0)),
            pl.BlockSpec((2, 1, 128), lambda i: (i, 0, 0)),
            pl.BlockSpec((2, 1, 128), lambda i: (i, 0, 0)),
        ],
        out_shape=[
            jax.ShapeDtypeStruct((B, 1, 128), jnp.float32),
            jax.ShapeDtypeStruct((B, 1, 128), jnp.float32),
            jax.ShapeDtypeStruct((B, 1, 128), jnp.int32),
        ],
    )(logits4, targets4)


def _sc_hist_body(l_hbm, t_hbm, top_hbm, out_hbm,
                  lb0, lb1, tb0, tb1, topv, h0, h1, h2, sem0, sem1):
    cid = lax.axis_index("c")
    sid = lax.axis_index("s")
    img = cid * 4 + sid // 4
    part = sid % 4
    orow = img * 4 + part
    band0 = part * (NCHUNK * 16)

    zeros16 = jnp.zeros((16,), jnp.float32)
    iota = lax.iota(jnp.int32, 16)
    nxt_idx = jnp.minimum(iota + 1, 15)
    not_last = iota < 15
    cnt_run = (iota + 1).astype(jnp.float32)
    neg_cnt_run = -cnt_run
    gdn = lax.GatherDimensionNumbers(
        offset_dims=(), collapsed_slice_dims=(0,), start_index_map=(0,))

    lbufs = (lb0, lb1)
    tbufs = (tb0, tb1)
    sems = (sem0, sem1)

    def _start(ci):
        rows = band0 + ci * 16
        lc = pltpu.async_copy(
            l_hbm.at[img, pl.ds(rows, 16), :], lbufs[ci % 2], sems[ci % 2])
        tc_ = pltpu.async_copy(
            t_hbm.at[img, pl.ds(rows, 16), :], tbufs[ci % 2], sems[ci % 2])
        return lc, tc_

    pend = _start(0)

    @plsc.parallel_loop(0, NBP // 16, unroll=8)
    def _zero(j):
        h0[pl.ds(j * 16, 16)] = zeros16
        h1[pl.ds(j * 16, 16)] = zeros16
        h2[pl.ds(j * 16, 16)] = zeros16

    pltpu.sync_copy(top_hbm.at[img, 0, pl.ds(0, 16)], topv)
    top = topv[...]

    for ci in range(NCHUNK):
        lbuf = lbufs[ci % 2]
        tbuf = tbufs[ci % 2]
        nxt_pend = _start(ci + 1) if ci + 1 < NCHUNK else None
        pend[0].wait()
        pend[1].wait()
        pend = nxt_pend

        @plsc.parallel_loop(0, CH // 16, unroll=4)
        def _vec(v):
            r = v >> 5
            c = (v & 31) * 16
            l = lbuf[r, pl.ds(c, 16)]
            t = tbuf[r, pl.ds(c, 16)]
            lt = l * t
            e = (1.0 + l) - (lt + lt)
            es0 = jnp.maximum(e, 0.0)
            ebits = lax.bitcast_convert_type(es0, jnp.int32)
            d = lax.shift_right_arithmetic(top - ebits, SHIFT)
            bkt = jnp.minimum(jnp.maximum((NB - 1) - d, 0), NB - 1)
            bs, es = plsc.sort_key_val(bkt, es0)
            _, gs = plsc.sort_key_val(bkt, t)
            nxt = lax.gather(bs, nxt_idx[:, None], gdn, slice_sizes=(1,),
                             mode=lax.GatherScatterMode.PROMISE_IN_BOUNDS)
            brk = bs != nxt
            isend = brk | (~not_last)
            issub = brk & not_last
            ce = plsc.cumsum(es)
            cg = plsc.cumsum(gs)
            plsc.addupdate_scatter(h0, [bs], cnt_run, mask=isend)
            plsc.addupdate_scatter(h0, [nxt], neg_cnt_run, mask=issub)
            plsc.addupdate_scatter(h1, [bs], cg, mask=isend)
            plsc.addupdate_scatter(h1, [nxt], -cg, mask=issub)
            plsc.addupdate_scatter(h2, [bs], ce, mask=isend)
            plsc.addupdate_scatter(h2, [nxt], -ce, mask=issub)

    obase = orow * 3 * NBP
    o0 = pltpu.async_copy(h0, out_hbm.at[pl.ds(obase, NBP)], sem0)
    o1 = pltpu.async_copy(h1, out_hbm.at[pl.ds(obase + NBP, NBP)], sem1)
    o2 = pltpu.async_copy(h2, out_hbm.at[pl.ds(obase + 2 * NBP, NBP)], sem0)
    o0.wait()
    o1.wait()
    o2.wait()


def _sc_hist(l3, t3, topflat):
    mesh = plsc.VectorSubcoreMesh(core_axis_name="c", subcore_axis_name="s")
    k = functools.partial(
        pl.kernel,
        mesh=mesh,
        compiler_params=pltpu.CompilerParams(needs_layout_passes=False),
        out_type=jax.ShapeDtypeStruct((NW * 3 * NBP,), jnp.float32),
        scratch_types=[
            pltpu.VMEM((16, 512), jnp.float32),
            pltpu.VMEM((16, 512), jnp.float32),
            pltpu.VMEM((16, 512), jnp.float32),
            pltpu.VMEM((16, 512), jnp.float32),
            pltpu.VMEM((16,), jnp.int32),
            pltpu.VMEM((NBP,), jnp.float32),
            pltpu.VMEM((NBP,), jnp.float32),
            pltpu.VMEM((NBP,), jnp.float32),
            pltpu.SemaphoreType.DMA,
            pltpu.SemaphoreType.DMA,
        ],
    )(_sc_hist_body)
    return k(l3, t3, topflat)


def _fin_body(h_ref, p_ref, m_ref, o_ref):
    f32 = jnp.float32
    r = lax.broadcasted_iota(jnp.int32, (128, 128), 0)
    c = lax.broadcasted_iota(jnp.int32, (128, 128), 1)
    w_incl = (r >= c).astype(f32)
    w_strict = (r > c).astype(f32)

    total = jnp.zeros((), f32)
    for img in range(B):
        r0 = img * 4
        cnt = (h_ref[r0, 0] + h_ref[r0 + 1, 0]) + (h_ref[r0 + 2, 0] + h_ref[r0 + 3, 0])
        cntp = (h_ref[r0, 1] + h_ref[r0 + 1, 1]) + (h_ref[r0 + 2, 1] + h_ref[r0 + 3, 1])
        sm = (h_ref[r0, 2] + h_ref[r0 + 1, 2]) + (h_ref[r0 + 2, 2] + h_ref[r0 + 3, 2])

        def suffix(x):
            s_in = jnp.dot(x, w_incl, preferred_element_type=f32)
            rowtot = s_in[:, :1]
            above = jnp.dot(w_strict.T, rowtot, preferred_element_type=f32)
            return s_in + above

        suf = suffix(cnt)
        sufp = suffix(cntp)
        a = suf - cnt
        ap = sufp - cntp
        p = p_ref[img, 0, 0]
        m = m_ref[img, 0, 0]
        tp = cntp
        tn = cnt - cntp
        sumpos = sm * cntp / jnp.maximum(cnt, 1.0)
        sumneg = sm - sumpos
        aa = p + a - ap
        contrib = (sumpos + sumneg * (p - ap - tp) / jnp.maximum(aa + tn, 1.0)
                   ) / jnp.maximum(aa, 1.0)
        loss = jnp.sum(contrib)
        loss = jnp.where(p == 0.0, jnp.maximum(m, 0.0), loss)
        total = total + loss
    o_ref[...] = jnp.broadcast_to(total * (1.0 / B), (1, 1))


def _finalize(h4, pc, mc):
    return pl.pallas_call(
        _fin_body,
        in_specs=[
            pl.BlockSpec((NW, 3, NBR, 128), lambda: (0, 0, 0, 0)),
            pl.BlockSpec((B, 1, 128), lambda: (0, 0, 0)),
            pl.BlockSpec((B, 1, 128), lambda: (0, 0, 0)),
        ],
        out_specs=pl.BlockSpec((1, 1), lambda: (0, 0)),
        out_shape=jax.ShapeDtypeStruct((1, 1), jnp.float32),
    )(h4, pc, mc)


def kernel(logits, targets):
    l4 = logits.reshape(B, 1, 512, 512)
    t4 = targets.reshape(B, 1, 512, 512)
    mc, pc, top = _prepass(l4, t4)
    hists = _sc_hist(logits.reshape(B, 512, 512), targets.reshape(B, 512, 512),
                     top)
    h4 = hists.reshape(NW, 3, NBR, 128)
    out = _finalize(h4, pc, mc)
    return out.reshape(())


# speedup vs baseline: 1.0835x; 1.0133x over previous
import functools

import jax
import jax.numpy as jnp
from jax import lax
from jax.experimental import pallas as pl
from jax.experimental.pallas import tpu as pltpu
from jax.experimental.pallas import tpu_sc as plsc

B = 8
N = 512 * 512
NB = 16384
NBR = 128
NBP = NBR * 128
SHIFT = 14
NW = 32
PER_W = N * B // NW
CH = 16384
NCHUNK = PER_W // CH


def _prep_body(l_ref, t_ref, m_ref, p_ref, top_ref):
    for i in range(2):
        l = l_ref[i, 0]
        t = t_ref[i, 0]
        tpos = jnp.where(t > 0.5, 1.0, 0.0).astype(jnp.float32)
        e = 1.0 - l * (2.0 * tpos - 1.0)
        m = jnp.max(e)
        m_ref[i, 0, :] = jnp.broadcast_to(m, (128,))
        p_ref[i, 0, :] = jnp.broadcast_to(jnp.sum(tpos), (128,))
        top_ref[i, 0, :] = jnp.broadcast_to(
            lax.bitcast_convert_type(m, jnp.int32), (128,))


def _prepass(logits4, targets4):
    return pl.pallas_call(
        _prep_body,
        grid=(B // 2,),
        in_specs=[
            pl.BlockSpec((2, 1, 512, 512), lambda i: (i, 0, 0, 0)),
            pl.BlockSpec((2, 1, 512, 512), lambda i: (i, 0, 0, 0)),
        ],
        out_specs=[
            pl.BlockSpec((2, 1, 128), lambda i: (i, 0, 0)),
            pl.BlockSpec((2, 1, 128), lambda i: (i, 0, 0)),
            pl.BlockSpec((2, 1, 128), lambda i: (i, 0, 0)),
        ],
        out_shape=[
            jax.ShapeDtypeStruct((B, 1, 128), jnp.float32),
            jax.ShapeDtypeStruct((B, 1, 128), jnp.float32),
            jax.ShapeDtypeStruct((B, 1, 128), jnp.int32),
        ],
    )(logits4, targets4)


def _sc_hist_body(l_hbm, t_hbm, top_hbm, out_hbm,
                  lb0, lb1, tb0, tb1, topv, h0, h1, h2, sem0, sem1):
    cid = lax.axis_index("c")
    sid = lax.axis_index("s")
    img = cid * 4 + sid // 4
    part = sid % 4
    orow = img * 4 + part
    band0 = part * (NCHUNK * 32)

    zeros16 = jnp.zeros((16,), jnp.float32)
    iota = lax.iota(jnp.int32, 16)
    nxt_idx = jnp.minimum(iota + 1, 15)
    not_last = iota < 15
    cnt_run = (iota + 1).astype(jnp.float32)
    neg_cnt_run = -cnt_run
    gdn = lax.GatherDimensionNumbers(
        offset_dims=(), collapsed_slice_dims=(0,), start_index_map=(0,))

    lbufs = (lb0, lb1)
    tbufs = (tb0, tb1)
    sems = (sem0, sem1)

    def _start(ci):
        rows = band0 + ci * 32
        lc = pltpu.async_copy(
            l_hbm.at[img, pl.ds(rows, 32), :], lbufs[ci % 2], sems[ci % 2])
        tc_ = pltpu.async_copy(
            t_hbm.at[img, pl.ds(rows, 32), :], tbufs[ci % 2], sems[ci % 2])
        return lc, tc_

    pend = _start(0)

    @plsc.parallel_loop(0, NBP // 16, unroll=8)
    def _zero(j):
        h0[pl.ds(j * 16, 16)] = zeros16
        h1[pl.ds(j * 16, 16)] = zeros16
        h2[pl.ds(j * 16, 16)] = zeros16

    pltpu.sync_copy(top_hbm.at[img, 0, pl.ds(0, 16)], topv)
    top = topv[...]

    for ci in range(NCHUNK):
        lbuf = lbufs[ci % 2]
        tbuf = tbufs[ci % 2]
        nxt_pend = _start(ci + 1) if ci + 1 < NCHUNK else None
        pend[0].wait()
        pend[1].wait()
        pend = nxt_pend

        @plsc.parallel_loop(0, CH // 16, unroll=4)
        def _vec(v):
            r = v >> 5
            c = (v & 31) * 16
            l = lbuf[r, pl.ds(c, 16)]
            t = tbuf[r, pl.ds(c, 16)]
            lt = l * t
            e = (1.0 + l) - (lt + lt)
            es0 = jnp.maximum(e, 0.0)
            ebits = lax.bitcast_convert_type(es0, jnp.int32)
            d = lax.shift_right_arithmetic(top - ebits, SHIFT)
            bkt = jnp.minimum(jnp.maximum((NB - 1) - d, 0), NB - 1)
            bs, es = plsc.sort_key_val(bkt, es0)
            _, gs = plsc.sort_key_val(bkt, t)
            nxt = lax.gather(bs, nxt_idx[:, None], gdn, slice_sizes=(1,),
                             mode=lax.GatherScatterMode.PROMISE_IN_BOUNDS)
            brk = bs != nxt
            isend = brk | (~not_last)
            issub = brk & not_last
            ce = plsc.cumsum(es)
            cg = plsc.cumsum(gs)
            plsc.addupdate_scatter(h0, [bs], cnt_run, mask=isend)
            plsc.addupdate_scatter(h0, [nxt], neg_cnt_run, mask=issub)
            plsc.addupdate_scatter(h1, [bs], cg, mask=isend)
            plsc.addupdate_scatter(h1, [nxt], -cg, mask=issub)
            plsc.addupdate_scatter(h2, [bs], ce, mask=isend)
            plsc.addupdate_scatter(h2, [nxt], -ce, mask=issub)

    obase = orow * 3 * NBP
    o0 = pltpu.async_copy(h0, out_hbm.at[pl.ds(obase, NBP)], sem0)
    o1 = pltpu.async_copy(h1, out_hbm.at[pl.ds(obase + NBP, NBP)], sem1)
    o2 = pltpu.async_copy(h2, out_hbm.at[pl.ds(obase + 2 * NBP, NBP)], sem0)
    o0.wait()
    o1.wait()
    o2.wait()


def _sc_hist(l3, t3, topflat):
    mesh = plsc.VectorSubcoreMesh(core_axis_name="c", subcore_axis_name="s")
    k = functools.partial(
        pl.kernel,
        mesh=mesh,
        compiler_params=pltpu.CompilerParams(needs_layout_passes=False),
        out_type=jax.ShapeDtypeStruct((NW * 3 * NBP,), jnp.float32),
        scratch_types=[
            pltpu.VMEM((32, 512), jnp.float32),
            pltpu.VMEM((32, 512), jnp.float32),
            pltpu.VMEM((32, 512), jnp.float32),
            pltpu.VMEM((32, 512), jnp.float32),
            pltpu.VMEM((16,), jnp.int32),
            pltpu.VMEM((NBP,), jnp.float32),
            pltpu.VMEM((NBP,), jnp.float32),
            pltpu.VMEM((NBP,), jnp.float32),
            pltpu.SemaphoreType.DMA,
            pltpu.SemaphoreType.DMA,
        ],
    )(_sc_hist_body)
    return k(l3, t3, topflat)


def _fin_body(h_ref, p_ref, m_ref, o_ref):
    f32 = jnp.float32
    r = lax.broadcasted_iota(jnp.int32, (128, 128), 0)
    c = lax.broadcasted_iota(jnp.int32, (128, 128), 1)
    w_incl = (r >= c).astype(f32)
    w_strict = (r > c).astype(f32)

    total = jnp.zeros((), f32)
    for img in range(B):
        r0 = img * 4
        cnt = (h_ref[r0, 0] + h_ref[r0 + 1, 0]) + (h_ref[r0 + 2, 0] + h_ref[r0 + 3, 0])
        cntp = (h_ref[r0, 1] + h_ref[r0 + 1, 1]) + (h_ref[r0 + 2, 1] + h_ref[r0 + 3, 1])
        sm = (h_ref[r0, 2] + h_ref[r0 + 1, 2]) + (h_ref[r0 + 2, 2] + h_ref[r0 + 3, 2])

        def suffix(x):
            s_in = jnp.dot(x, w_incl, preferred_element_type=f32)
            rowtot = s_in[:, :1]
            above = jnp.dot(w_strict.T, rowtot, preferred_element_type=f32)
            return s_in + above

        suf = suffix(cnt)
        sufp = suffix(cntp)
        a = suf - cnt
        ap = sufp - cntp
        p = p_ref[img, 0, 0]
        m = m_ref[img, 0, 0]
        tp = cntp
        tn = cnt - cntp
        sumpos = sm * cntp / jnp.maximum(cnt, 1.0)
        sumneg = sm - sumpos
        aa = p + a - ap
        contrib = (sumpos + sumneg * (p - ap - tp) / jnp.maximum(aa + tn, 1.0)
                   ) / jnp.maximum(aa, 1.0)
        loss = jnp.sum(contrib)
        loss = jnp.where(p == 0.0, jnp.maximum(m, 0.0), loss)
        total = total + loss
    o_ref[...] = jnp.broadcast_to(total * (1.0 / B), (1, 1))


def _finalize(h4, pc, mc):
    return pl.pallas_call(
        _fin_body,
        in_specs=[
            pl.BlockSpec((NW, 3, NBR, 128), lambda: (0, 0, 0, 0)),
            pl.BlockSpec((B, 1, 128), lambda: (0, 0, 0)),
            pl.BlockSpec((B, 1, 128), lambda: (0, 0, 0)),
        ],
        out_specs=pl.BlockSpec((1, 1), lambda: (0, 0)),
        out_shape=jax.ShapeDtypeStruct((1, 1), jnp.float32),
    )(h4, pc, mc)


def kernel(logits, targets):
    l4 = logits.reshape(B, 1, 512, 512)
    t4 = targets.reshape(B, 1, 512, 512)
    mc, pc, top = _prepass(l4, t4)
    hists = _sc_hist(logits.reshape(B, 512, 512), targets.reshape(B, 512, 512),
                     top)
    h4 = hists.reshape(NW, 3, NBR, 128)
    out = _finalize(h4, pc, mc)
    return out.reshape(())
